# baseline (speedup 1.0000x reference)
import functools
import jax
import jax.numpy as jnp
from jax import lax
from jax.experimental import pallas as pl
from jax.experimental.pallas import tpu as pltpu
from jax.experimental.pallas import tpu_sc as plsc

_ROWS = 16384
_COLS = 1024
_BLK = 1024
_NC, _NS, _L = 1, 16, 16
_NW = _NC * _NS
_CHUNK = _ROWS // _NW
_TBL = 64

_sc_mesh = plsc.VectorSubcoreMesh(
    core_axis_name="c", subcore_axis_name="s", num_cores=_NC, num_subcores=_NS
)


@functools.partial(
    pl.kernel,
    out_type=[
        jax.ShapeDtypeStruct((_ROWS,), jnp.float32),
        jax.ShapeDtypeStruct((_ROWS,), jnp.float32),
    ],
    mesh=_sc_mesh,
    scratch_types=[
        pltpu.VMEM((_CHUNK,), jnp.int32),
        pltpu.VMEM((2 * _TBL,), jnp.float32),
        pltpu.VMEM((_CHUNK,), jnp.float32),
        pltpu.VMEM((_CHUNK,), jnp.float32),
        pltpu.SemaphoreType.DMA,
        pltpu.SemaphoreType.DMA,
    ],
    compiler_params=pltpu.CompilerParams(needs_layout_passes=False),
)
def _sc_gather(t_hbm, tab_hbm, a_hbm, b_hbm, t_v, tab_v, a_v, b_v, sem1, sem2):
    wid = lax.axis_index("s") * _NC + lax.axis_index("c")
    base = wid * _CHUNK
    cp_t = pltpu.async_copy(t_hbm.at[pl.ds(base, _CHUNK)], t_v, sem1)
    cp_tab = pltpu.async_copy(tab_hbm, tab_v, sem2)
    cp_t.wait()
    cp_tab.wait()

    for i in range(_CHUNK // _L):
        idx = t_v[pl.ds(i * _L, _L)]
        a_v[pl.ds(i * _L, _L)] = plsc.load_gather(tab_v, [idx])
        b_v[pl.ds(i * _L, _L)] = plsc.load_gather(tab_v, [idx + _TBL])

    cp_a = pltpu.async_copy(a_v, a_hbm.at[pl.ds(base, _CHUNK)], sem1)
    cp_b = pltpu.async_copy(b_v, b_hbm.at[pl.ds(base, _CHUNK)], sem2)
    cp_a.wait()
    cp_b.wait()


def _fma_kernel(a_ref, b_ref, x0_ref, eps_ref, o_ref):
    a = a_ref[0, 0, :]
    b = b_ref[0, 0, :]
    o_ref[...] = a[:, None] * x0_ref[...] + b[:, None] * eps_ref[...]


def kernel(x0, eps, t, alpha_hat):
    t32 = t.astype(jnp.int32)
    nb = alpha_hat.shape[0]
    tab_sc = jnp.concatenate(
        [
            jnp.pad(jnp.sqrt(alpha_hat), (0, _TBL - nb)),
            jnp.pad(jnp.sqrt(1.0 - alpha_hat), (0, _TBL - nb)),
        ]
    )

    a, b = _sc_gather(t32, tab_sc)
    a3 = a.reshape(_ROWS // _BLK, 1, _BLK)
    b3 = b.reshape(_ROWS // _BLK, 1, _BLK)

    return pl.pallas_call(
        _fma_kernel,
        grid=(_ROWS // _BLK,),
        in_specs=[
            pl.BlockSpec((1, 1, _BLK), lambda i: (i, 0, 0)),
            pl.BlockSpec((1, 1, _BLK), lambda i: (i, 0, 0)),
            pl.BlockSpec((_BLK, _COLS), lambda i: (i, 0)),
            pl.BlockSpec((_BLK, _COLS), lambda i: (i, 0)),
        ],
        out_specs=pl.BlockSpec((_BLK, _COLS), lambda i: (i, 0)),
        out_shape=jax.ShapeDtypeStruct((_ROWS, _COLS), jnp.float32),
    )(a3, b3, x0, eps)


# speedup vs baseline: 1.0082x; 1.0082x over previous
import functools
import jax
import jax.numpy as jnp
from jax import lax
from jax.experimental import pallas as pl
from jax.experimental.pallas import tpu as pltpu
from jax.experimental.pallas import tpu_sc as plsc

_ROWS = 16384
_COLS = 1024
_BLK = 1024
_NC, _NS, _L = 1, 16, 16
_NW = _NC * _NS
_CHUNK = _ROWS // _NW
_TBL = 64

_sc_mesh = plsc.VectorSubcoreMesh(
    core_axis_name="c", subcore_axis_name="s", num_cores=_NC, num_subcores=_NS
)


@functools.partial(
    pl.kernel,
    out_type=jax.ShapeDtypeStruct((_ROWS,), jnp.float32),
    mesh=_sc_mesh,
    scratch_types=[
        pltpu.VMEM((_CHUNK,), jnp.int32),
        pltpu.VMEM((_TBL,), jnp.float32),
        pltpu.VMEM((_CHUNK,), jnp.float32),
        pltpu.SemaphoreType.DMA,
        pltpu.SemaphoreType.DMA,
    ],
    compiler_params=pltpu.CompilerParams(
        needs_layout_passes=False, skip_device_barrier=True
    ),
)
def _sc_gather(t_hbm, tab_hbm, a_hbm, t_v, tab_v, a_v, sem1, sem2):
    wid = lax.axis_index("s") * _NC + lax.axis_index("c")
    base = wid * _CHUNK
    cp_t = pltpu.async_copy(t_hbm.at[pl.ds(base, _CHUNK)], t_v, sem1)
    cp_tab = pltpu.async_copy(tab_hbm, tab_v, sem2)
    cp_t.wait()
    cp_tab.wait()

    for i in range(_CHUNK // _L):
        idx = t_v[pl.ds(i * _L, _L)]
        a_v[pl.ds(i * _L, _L)] = plsc.load_gather(tab_v, [idx])

    pltpu.async_copy(a_v, a_hbm.at[pl.ds(base, _CHUNK)], sem1).wait()


def _fma_kernel(a_ref, x0_ref, eps_ref, o_ref):
    a = a_ref[0, 0, :]
    b = jnp.sqrt(jnp.maximum(1.0 - a * a, 0.0))
    o_ref[...] = a[:, None] * x0_ref[...] + b[:, None] * eps_ref[...]


def kernel(x0, eps, t, alpha_hat):
    t32 = t.astype(jnp.int32)
    nb = alpha_hat.shape[0]
    tab_sc = jnp.pad(jnp.sqrt(alpha_hat), (0, _TBL - nb))

    a = _sc_gather(t32, tab_sc)
    a3 = a.reshape(_ROWS // _BLK, 1, _BLK)

    return pl.pallas_call(
        _fma_kernel,
        grid=(_ROWS // _BLK,),
        in_specs=[
            pl.BlockSpec((1, 1, _BLK), lambda i: (i, 0, 0)),
            pl.BlockSpec((_BLK, _COLS), lambda i: (i, 0)),
            pl.BlockSpec((_BLK, _COLS), lambda i: (i, 0)),
        ],
        out_specs=pl.BlockSpec((_BLK, _COLS), lambda i: (i, 0)),
        out_shape=jax.ShapeDtypeStruct((_ROWS, _COLS), jnp.float32),
    )(a3, x0, eps)


# speedup vs baseline: 1.0378x; 1.0294x over previous
import functools
import jax
import jax.numpy as jnp
from jax import lax
from jax.experimental import pallas as pl
from jax.experimental.pallas import tpu as pltpu
from jax.experimental.pallas import tpu_sc as plsc

_ROWS = 16384
_COLS = 1024
_BLK = 1024
_HEAD = 8192
_TAIL = _ROWS - _HEAD
_HB = _HEAD // _BLK
_TB = _TAIL // _BLK
_NC, _NS, _L = 1, 16, 16
_NW = _NC * _NS
_CHUNK = _TAIL // _NW
_TBL = 64
_TPAD = 128

_sc_mesh = plsc.VectorSubcoreMesh(
    core_axis_name="c", subcore_axis_name="s", num_cores=_NC, num_subcores=_NS
)


@functools.partial(
    pl.kernel,
    out_type=jax.ShapeDtypeStruct((_TAIL,), jnp.float32),
    mesh=_sc_mesh,
    scratch_types=[
        pltpu.VMEM((_CHUNK,), jnp.int32),
        pltpu.VMEM((_TBL,), jnp.float32),
        pltpu.VMEM((_CHUNK,), jnp.float32),
        pltpu.SemaphoreType.DMA,
        pltpu.SemaphoreType.DMA,
    ],
    compiler_params=pltpu.CompilerParams(
        needs_layout_passes=False, skip_device_barrier=True
    ),
)
def _sc_gather(t_hbm, tab_hbm, a_hbm, t_v, tab_v, a_v, sem1, sem2):
    wid = lax.axis_index("s") * _NC + lax.axis_index("c")
    base = wid * _CHUNK
    cp_t = pltpu.async_copy(t_hbm.at[pl.ds(_HEAD + base, _CHUNK)], t_v, sem1)
    cp_tab = pltpu.async_copy(tab_hbm, tab_v, sem2)
    cp_t.wait()
    cp_tab.wait()

    for i in range(_CHUNK // _L):
        idx = t_v[pl.ds(i * _L, _L)]
        a_v[pl.ds(i * _L, _L)] = plsc.load_gather(tab_v, [idx])

    pltpu.async_copy(a_v, a_hbm.at[pl.ds(base, _CHUNK)], sem1).wait()


def _head_kernel(t_ref, sa_ref, sb_ref, x0_ref, eps_ref, o_ref):
    t_blk = t_ref[0, 0, :]
    cols = jax.lax.broadcasted_iota(jnp.int32, (t_blk.shape[0], _TPAD), 1)
    onehot = t_blk[:, None] == cols
    a = jnp.sum(jnp.where(onehot, sa_ref[0, :][None, :], 0.0), axis=1)
    b = jnp.sum(jnp.where(onehot, sb_ref[0, :][None, :], 0.0), axis=1)
    o_ref[...] = a[:, None] * x0_ref[...] + b[:, None] * eps_ref[...]


def _tail_kernel(part_ref, a_ref, x0_ref, eps_ref, o_ref):
    del part_ref
    a = a_ref[0, 0, :]
    b = jnp.sqrt(jnp.maximum(1.0 - a * a, 0.0))
    o_ref[...] = a[:, None] * x0_ref[...] + b[:, None] * eps_ref[...]


def kernel(x0, eps, t, alpha_hat):
    t32 = t.astype(jnp.int32)
    nb = alpha_hat.shape[0]
    tab_sc = jnp.pad(jnp.sqrt(alpha_hat), (0, _TBL - nb))
    sa_tc = jnp.pad(jnp.sqrt(alpha_hat), (0, _TPAD - nb)).reshape(1, _TPAD)
    sb_tc = jnp.pad(jnp.sqrt(1.0 - alpha_hat), (0, _TPAD - nb)).reshape(1, _TPAD)

    t3 = t32.reshape(_ROWS // _BLK, 1, _BLK)

    a = _sc_gather(t32, tab_sc)
    a3 = a.reshape(_TB, 1, _BLK)

    partial_out = pl.pallas_call(
        _head_kernel,
        grid=(_HB,),
        in_specs=[
            pl.BlockSpec((1, 1, _BLK), lambda i: (i, 0, 0)),
            pl.BlockSpec((1, _TPAD), lambda i: (0, 0)),
            pl.BlockSpec((1, _TPAD), lambda i: (0, 0)),
            pl.BlockSpec((_BLK, _COLS), lambda i: (i, 0)),
            pl.BlockSpec((_BLK, _COLS), lambda i: (i, 0)),
        ],
        out_specs=pl.BlockSpec((_BLK, _COLS), lambda i: (i, 0)),
        out_shape=jax.ShapeDtypeStruct((_ROWS, _COLS), jnp.float32),
    )(t3, sa_tc, sb_tc, x0, eps)

    return pl.pallas_call(
        _tail_kernel,
        grid=(_TB,),
        in_specs=[
            pl.BlockSpec(memory_space=pl.ANY),
            pl.BlockSpec((1, 1, _BLK), lambda j: (j, 0, 0)),
            pl.BlockSpec((_BLK, _COLS), lambda j: (_HB + j, 0)),
            pl.BlockSpec((_BLK, _COLS), lambda j: (_HB + j, 0)),
        ],
        out_specs=pl.BlockSpec((_BLK, _COLS), lambda j: (_HB + j, 0)),
        out_shape=jax.ShapeDtypeStruct((_ROWS, _COLS), jnp.float32),
        input_output_aliases={0: 0},
    )(partial_out, a3, x0, eps)


# speedup vs baseline: 1.0417x; 1.0037x over previous
import functools
import jax
import jax.numpy as jnp
from jax import lax
from jax.experimental import pallas as pl
from jax.experimental.pallas import tpu as pltpu
from jax.experimental.pallas import tpu_sc as plsc

_ROWS = 16384
_COLS = 1024
_BLK = 1024
_HEAD = 8192
_TAIL = _ROWS - _HEAD
_HB = _HEAD // _BLK
_TB = _TAIL // _BLK
_NC, _NS, _L = 1, 16, 16
_NW = _NC * _NS
_CHUNK = _TAIL // _NW
_TBL = 64
_TPAD = 128

_sc_mesh = plsc.VectorSubcoreMesh(
    core_axis_name="c", subcore_axis_name="s", num_cores=_NC, num_subcores=_NS
)


@functools.partial(
    pl.kernel,
    out_type=[
        jax.ShapeDtypeStruct((_TAIL,), jnp.float32),
        jax.ShapeDtypeStruct((_TAIL,), jnp.float32),
    ],
    mesh=_sc_mesh,
    scratch_types=[
        pltpu.VMEM((_CHUNK,), jnp.int32),
        pltpu.VMEM((2 * _TBL,), jnp.float32),
        pltpu.VMEM((_CHUNK,), jnp.float32),
        pltpu.VMEM((_CHUNK,), jnp.float32),
        pltpu.SemaphoreType.DMA,
        pltpu.SemaphoreType.DMA,
    ],
    compiler_params=pltpu.CompilerParams(needs_layout_passes=False),
)
def _sc_gather(t_hbm, tab_hbm, a_hbm, b_hbm, t_v, tab_v, a_v, b_v, sem1, sem2):
    wid = lax.axis_index("s") * _NC + lax.axis_index("c")
    base = wid * _CHUNK
    cp_t = pltpu.async_copy(t_hbm.at[pl.ds(_HEAD + base, _CHUNK)], t_v, sem1)
    cp_tab = pltpu.async_copy(tab_hbm, tab_v, sem2)
    cp_t.wait()
    cp_tab.wait()

    for i in range(_CHUNK // _L):
        idx = t_v[pl.ds(i * _L, _L)]
        a_v[pl.ds(i * _L, _L)] = plsc.load_gather(tab_v, [idx])
        b_v[pl.ds(i * _L, _L)] = plsc.load_gather(tab_v, [idx + _TBL])

    cp_a = pltpu.async_copy(a_v, a_hbm.at[pl.ds(base, _CHUNK)], sem1)
    cp_b = pltpu.async_copy(b_v, b_hbm.at[pl.ds(base, _CHUNK)], sem2)
    cp_a.wait()
    cp_b.wait()


def _head_kernel(t_ref, sa_ref, sb_ref, x0_ref, eps_ref, o_ref):
    t_blk = t_ref[0, 0, :]
    cols = jax.lax.broadcasted_iota(jnp.int32, (t_blk.shape[0], _TPAD), 1)
    onehot = t_blk[:, None] == cols
    a = jnp.sum(jnp.where(onehot, sa_ref[0, :][None, :], 0.0), axis=1)
    b = jnp.sum(jnp.where(onehot, sb_ref[0, :][None, :], 0.0), axis=1)
    o_ref[...] = a[:, None] * x0_ref[...] + b[:, None] * eps_ref[...]


def _tail_kernel(part_ref, a_ref, b_ref, x0_ref, eps_ref, o_ref):
    del part_ref
    a = a_ref[0, 0, :]
    b = b_ref[0, 0, :]
    o_ref[...] = a[:, None] * x0_ref[...] + b[:, None] * eps_ref[...]


def kernel(x0, eps, t, alpha_hat):
    t32 = t.astype(jnp.int32)
    nb = alpha_hat.shape[0]
    tab_sc = jnp.concatenate(
        [
            jnp.pad(jnp.sqrt(alpha_hat), (0, _TBL - nb)),
            jnp.pad(jnp.sqrt(1.0 - alpha_hat), (0, _TBL - nb)),
        ]
    )
    sa_tc = jnp.pad(jnp.sqrt(alpha_hat), (0, _TPAD - nb)).reshape(1, _TPAD)
    sb_tc = jnp.pad(jnp.sqrt(1.0 - alpha_hat), (0, _TPAD - nb)).reshape(1, _TPAD)

    t3 = t32.reshape(_ROWS // _BLK, 1, _BLK)

    a, b = _sc_gather(t32, tab_sc)
    a3 = a.reshape(_TB, 1, _BLK)
    b3 = b.reshape(_TB, 1, _BLK)

    partial_out = pl.pallas_call(
        _head_kernel,
        grid=(_HB,),
        in_specs=[
            pl.BlockSpec((1, 1, _BLK), lambda i: (i, 0, 0)),
            pl.BlockSpec((1, _TPAD), lambda i: (0, 0)),
            pl.BlockSpec((1, _TPAD), lambda i: (0, 0)),
            pl.BlockSpec((_BLK, _COLS), lambda i: (i, 0)),
            pl.BlockSpec((_BLK, _COLS), lambda i: (i, 0)),
        ],
        out_specs=pl.BlockSpec((_BLK, _COLS), lambda i: (i, 0)),
        out_shape=jax.ShapeDtypeStruct((_ROWS, _COLS), jnp.float32),
    )(t3, sa_tc, sb_tc, x0, eps)

    return pl.pallas_call(
        _tail_kernel,
        grid=(_TB,),
        in_specs=[
            pl.BlockSpec(memory_space=pl.ANY),
            pl.BlockSpec((1, 1, _BLK), lambda j: (j, 0, 0)),
            pl.BlockSpec((1, 1, _BLK), lambda j: (j, 0, 0)),
            pl.BlockSpec((_BLK, _COLS), lambda j: (_HB + j, 0)),
            pl.BlockSpec((_BLK, _COLS), lambda j: (_HB + j, 0)),
        ],
        out_specs=pl.BlockSpec((_BLK, _COLS), lambda j: (_HB + j, 0)),
        out_shape=jax.ShapeDtypeStruct((_ROWS, _COLS), jnp.float32),
        input_output_aliases={0: 0},
    )(partial_out, a3, b3, x0, eps)
